# baseline (speedup 1.0000x reference)
import jax
import jax.numpy as jnp
from jax import lax
from jax.experimental import pallas as pl
from jax.experimental.pallas import tpu as pltpu
from jax.experimental.pallas import tpu_sc as plsc

_N = 10000
_E = 320000
_DIM = 128
_HEADS = 4
_DH = _DIM // _HEADS
_SCALE = _DH ** -0.5

_NC = 2
_NS = 16
_NCHUNK = 2
_EC = _E // _NCHUNK
_EPC = _EC // _NC
_EPT = _EPC // _NS
_C1 = 200
_N1 = _EPT // _C1
_C2 = 40
_N2 = _EPT // _C2
_RPT = 624
_TAIL = _N - _RPT * _NS
_DPT = 2496
_DTAIL = _N * _HEADS - _DPT * _NS

_MESH = plsc.VectorSubcoreMesh(
    core_axis_name="c", subcore_axis_name="s", num_cores=_NC, num_subcores=_NS
)


def _mm_t(x, w):
    return lax.dot_general(x, w, (((1,), (1,)), ((), ())),
                           preferred_element_type=jnp.float32)


def _tc0_body(x_ref, w_ref, b_ref, o_ref):
    o_ref[...] = _mm_t(x_ref[...], w_ref[...]) + b_ref[...]


def _tc0(q_nodes, Wq, bq):
    T = 2000
    return pl.pallas_call(
        _tc0_body,
        grid=(_N // T,),
        in_specs=[
            pl.BlockSpec((T, _DIM), lambda i: (i, 0)),
            pl.BlockSpec((_DIM, _DIM), lambda i: (0, 0)),
            pl.BlockSpec((1, _DIM), lambda i: (0, 0)),
        ],
        out_specs=pl.BlockSpec((T, _DIM), lambda i: (i, 0)),
        out_shape=jax.ShapeDtypeStruct((_N, _DIM), jnp.float32),
    )(q_nodes, Wq, bq.reshape(1, _DIM))


def _make_sc1(e0):
    def body(qp_hbm, dst_hbm, qg_hbm, idx0, idx1, qg0, qg1,
             is0, is1, gs0, gs1, ss0, ss1):
        c = lax.axis_index("c")
        s = lax.axis_index("s")
        src0 = e0 + c * _EPC + s * _EPT
        out0 = c * _EPC + s * _EPT
        idx = (idx0, idx1)
        qg = (qg0, qg1)
        isem = (is0, is1)
        gsem = (gs0, gs1)
        ssem = (ss0, ss1)

        def step(i, b):
            o = 1 - b
            pltpu.make_async_copy(dst_hbm.at[pl.ds(src0 + i * _C1, _C1)],
                                  idx[b], isem[b]).wait()

            @pl.when(i >= 2)
            def _():
                pltpu.make_async_copy(
                    qg[b], qg_hbm.at[pl.ds(out0 + i * _C1, _C1)],
                    ssem[b]).wait()

            pltpu.async_copy(qp_hbm.at[idx[b]], qg[b], gsem[b])

            @pl.when(i >= 1)
            def _():
                pb = out0 + (i - 1) * _C1
                pltpu.make_async_copy(qp_hbm.at[idx[o]], qg[o],
                                      gsem[o]).wait()
                pltpu.async_copy(qg[o], qg_hbm.at[pl.ds(pb, _C1)], ssem[o])

                @pl.when(i + 1 < _N1)
                def _():
                    pltpu.async_copy(
                        dst_hbm.at[pl.ds(src0 + (i + 1) * _C1, _C1)],
                        idx[o], isem[o])

        for b in (0, 1):
            pltpu.async_copy(dst_hbm.at[pl.ds(src0 + b * _C1, _C1)], idx[b],
                             isem[b])

        def pair(g, carry):
            for b in (0, 1):
                step(2 * g + b, b)
            return carry

        lax.fori_loop(0, _N1 // 2, pair, 0)
        bl = (_N1 - 1) % 2
        if _N1 % 2:
            step(_N1 - 1, bl)
        lb = out0 + (_N1 - 1) * _C1
        pltpu.make_async_copy(qp_hbm.at[idx[bl]], qg[bl], gsem[bl]).wait()
        pltpu.async_copy(qg[bl], qg_hbm.at[pl.ds(lb, _C1)], ssem[bl])
        pltpu.make_async_copy(qg[0], qg_hbm.at[pl.ds(lb, _C1)], ssem[0]).wait()
        pltpu.make_async_copy(qg[1], qg_hbm.at[pl.ds(lb, _C1)], ssem[1]).wait()

    return pl.kernel(
        body,
        out_type=jax.ShapeDtypeStruct((_EC, _DIM), jnp.float32),
        mesh=_MESH,
        scratch_types=[
            pltpu.VMEM((_C1,), jnp.int32),
            pltpu.VMEM((_C1,), jnp.int32),
            pltpu.VMEM((_C1, _DIM), jnp.float32),
            pltpu.VMEM((_C1, _DIM), jnp.float32),
        ] + [pltpu.SemaphoreType.DMA] * 6,
    )


def _tc1_body(k_ref, v_ref, qg_ref, wk_ref, bk_ref, wv_ref, bv_ref, m_ref,
              w_ref, u_ref):
    kp = _mm_t(k_ref[...], wk_ref[...]) + bk_ref[...]
    s = jnp.dot(qg_ref[...] * kp, m_ref[...],
                preferred_element_type=jnp.float32) * _SCALE
    u = jnp.exp(s)
    vp = _mm_t(v_ref[...], wv_ref[...]) + bv_ref[...]
    w_ref[...] = vp * _mm_t(u, m_ref[...])
    u_ref[...] = u


def _tc1(k_edges, v_edges, qg, Wk, bk, Wv, bv, mask, e0):
    T = 4000
    off = e0 // T
    full = lambda i: (0, 0)
    gblk = lambda i: (i + off, 0)
    blk = lambda i: (i, 0)
    return pl.pallas_call(
        _tc1_body,
        grid=(_EC // T,),
        in_specs=[
            pl.BlockSpec((T, _DIM), gblk),
            pl.BlockSpec((T, _DIM), gblk),
            pl.BlockSpec((T, _DIM), blk),
            pl.BlockSpec((_DIM, _DIM), full),
            pl.BlockSpec((1, _DIM), full),
            pl.BlockSpec((_DIM, _DIM), full),
            pl.BlockSpec((1, _DIM), full),
            pl.BlockSpec((_DIM, _HEADS), full),
        ],
        out_specs=[
            pl.BlockSpec((T, _DIM), blk),
            pl.BlockSpec((T, _HEADS), blk),
        ],
        out_shape=[
            jax.ShapeDtypeStruct((_EC, _DIM), jnp.float32),
            jax.ShapeDtypeStruct((_EC, _HEADS), jnp.float32),
        ],
    )(k_edges, v_edges, qg, Wk, bk.reshape(1, _DIM), Wv, bv.reshape(1, _DIM),
      mask)


def _make_sc2(e0):
    def body(w_hbm, uf_hbm, dst_hbm, i4_hbm, zag_hbm, zden_hbm, agg_out,
             den_out, agg_sh, den_sh, idx0, idx1, w0, w1, u0, u1, q0, q1,
             d_v, dt_v, ls0, ls1):
        c = lax.axis_index("c")
        s = lax.axis_index("s")
        idx = (idx0, idx1)
        wv = (w0, w1)
        uv = (u0, u1)
        qv = (q0, q1)
        lsem = (ls0, ls1)

        @pl.when(s == 0)
        def _():
            pltpu.sync_copy(zag_hbm, agg_sh)
            pltpu.sync_copy(zden_hbm, den_sh)

        plsc.subcore_barrier()
        src0 = e0 + c * _EPC + s * _EPT
        loc0 = c * _EPC + s * _EPT

        def issue(i, b):
            sb = src0 + i * _C2
            lb = loc0 + i * _C2
            pltpu.async_copy(dst_hbm.at[pl.ds(sb, _C2)], idx[b], lsem[b])
            pltpu.async_copy(w_hbm.at[pl.ds(lb, _C2)], wv[b], lsem[b])
            pltpu.async_copy(uf_hbm.at[pl.ds(lb * _HEADS, _C2 * _HEADS)],
                             uv[b], lsem[b])
            pltpu.async_copy(i4_hbm.at[pl.ds(sb * _HEADS, _C2 * _HEADS)],
                             qv[b], lsem[b])

        def drain(i, b):
            sb = src0 + i * _C2
            lb = loc0 + i * _C2
            pltpu.make_async_copy(dst_hbm.at[pl.ds(sb, _C2)], idx[b],
                                  lsem[b]).wait()
            pltpu.make_async_copy(w_hbm.at[pl.ds(lb, _C2)], wv[b],
                                  lsem[b]).wait()
            pltpu.make_async_copy(
                uf_hbm.at[pl.ds(lb * _HEADS, _C2 * _HEADS)], uv[b],
                lsem[b]).wait()
            pltpu.make_async_copy(
                i4_hbm.at[pl.ds(sb * _HEADS, _C2 * _HEADS)], qv[b],
                lsem[b]).wait()

        def scat(b):
            pltpu.sync_copy(wv[b], agg_sh.at[idx[b]], add=True)
            pltpu.sync_copy(uv[b], den_sh.at[qv[b]], add=True)

        for b in (0, 1):
            issue(b, b)

        def pair(g, carry):
            for b in (0, 1):
                i = 2 * g + b
                drain(i, b)
                scat(b)

                @pl.when(i + 2 < _N2)
                def _():
                    issue(i + 2, b)
            return carry

        lax.fori_loop(0, _N2 // 2, pair, 0)
        if _N2 % 2:
            drain(_N2 - 1, 0)
            scat(0)

        plsc.subcore_barrier()
        r0 = s * _RPT
        pltpu.sync_copy(agg_sh.at[pl.ds(r0, _RPT)],
                        agg_out.at[c, pl.ds(r0, _RPT)])
        d0 = s * _DPT
        pltpu.sync_copy(den_sh.at[pl.ds(d0, _DPT)], d_v)
        pltpu.sync_copy(d_v, den_out.at[pl.ds(c * _N * _HEADS + d0, _DPT)])

        @pl.when(s == 0)
        def _():
            t0 = _RPT * _NS
            pltpu.sync_copy(agg_sh.at[pl.ds(t0, _TAIL)],
                            agg_out.at[c, pl.ds(t0, _TAIL)])
            dt0 = _DPT * _NS
            pltpu.sync_copy(den_sh.at[pl.ds(dt0, _DTAIL)], dt_v)
            pltpu.sync_copy(dt_v,
                            den_out.at[pl.ds(c * _N * _HEADS + dt0, _DTAIL)])

    return pl.kernel(
        body,
        out_type=(
            jax.ShapeDtypeStruct((_NC, _N, _DIM), jnp.float32),
            jax.ShapeDtypeStruct((_NC * _N * _HEADS,), jnp.float32),
        ),
        mesh=_MESH,
        scratch_types=[
            pltpu.VMEM_SHARED((_N, _DIM), jnp.float32),
            pltpu.VMEM_SHARED((_N * _HEADS,), jnp.float32),
            pltpu.VMEM((_C2,), jnp.int32),
            pltpu.VMEM((_C2,), jnp.int32),
            pltpu.VMEM((_C2, _DIM), jnp.float32),
            pltpu.VMEM((_C2, _DIM), jnp.float32),
            pltpu.VMEM((_C2 * _HEADS,), jnp.float32),
            pltpu.VMEM((_C2 * _HEADS,), jnp.float32),
            pltpu.VMEM((_C2 * _HEADS,), jnp.int32),
            pltpu.VMEM((_C2 * _HEADS,), jnp.int32),
            pltpu.VMEM((_DPT,), jnp.float32),
            pltpu.VMEM((_DTAIL,), jnp.float32),
        ] + [pltpu.SemaphoreType.DMA] * 2,
    )


def _tc2_body(a0_ref, a1_ref, d0_ref, d1_ref, wo_ref, bo_ref, m_ref, o_ref):
    agg = a0_ref[0] + a0_ref[1] + a1_ref[0] + a1_ref[1]
    den = d0_ref[0] + d0_ref[1] + d1_ref[0] + d1_ref[1]
    d128 = _mm_t(den, m_ref[...])
    nrm = agg / (d128 + 1e-16)
    o_ref[...] = _mm_t(nrm, wo_ref[...]) + bo_ref[...]


def _tc2(agg_a, agg_b, den_a, den_b, Wo, bo, mask):
    B = 1000
    full = lambda i: (0, 0)
    pblk = lambda i: (0, i, 0)
    return pl.pallas_call(
        _tc2_body,
        grid=(_N // B,),
        in_specs=[
            pl.BlockSpec((_NC, B, _DIM), pblk),
            pl.BlockSpec((_NC, B, _DIM), pblk),
            pl.BlockSpec((_NC, B, _HEADS), pblk),
            pl.BlockSpec((_NC, B, _HEADS), pblk),
            pl.BlockSpec((_DIM, _DIM), full),
            pl.BlockSpec((1, _DIM), full),
            pl.BlockSpec((_DIM, _HEADS), full),
        ],
        out_specs=pl.BlockSpec((B, _DIM), lambda i: (i, 0)),
        out_shape=jax.ShapeDtypeStruct((_N, _DIM), jnp.float32),
    )(agg_a, agg_b, den_a.reshape(_NC, _N, _HEADS),
      den_b.reshape(_NC, _N, _HEADS), Wo, bo.reshape(1, _DIM), mask)


def kernel(q_nodes, k_edges, v_edges, edge_index, Wq, bq, Wk, bk, Wv, bv, Wo,
           bo):
    dst = edge_index[0]
    idx4 = dst[:, None] * _HEADS + jnp.arange(_HEADS, dtype=jnp.int32)[None, :]
    idx4 = idx4.reshape(_E * _HEADS)
    zag = jnp.zeros((_N, _DIM), jnp.float32)
    zden = jnp.zeros((_N * _HEADS,), jnp.float32)
    mask = (jnp.arange(_DIM)[:, None] // _DH
            == jnp.arange(_HEADS)[None, :]).astype(jnp.float32)

    qp = _tc0(q_nodes, Wq, bq)
    qg_a = _make_sc1(0)(qp, dst)
    qg_b = _make_sc1(_EC)(qp, dst)
    w_a, u_a = _tc1(k_edges, v_edges, qg_a, Wk, bk, Wv, bv, mask, 0)
    w_b, u_b = _tc1(k_edges, v_edges, qg_b, Wk, bk, Wv, bv, mask, _EC)
    agg_a, den_a = _make_sc2(0)(w_a, u_a.reshape(_EC * _HEADS), dst, idx4,
                                zag, zden)
    agg_b, den_b = _make_sc2(_EC)(w_b, u_b.reshape(_EC * _HEADS), dst, idx4,
                                  zag, zden)
    return _tc2(agg_a, agg_b, den_a, den_b, Wo, bo, mask)


# speedup vs baseline: 1.0530x; 1.0530x over previous
import jax
import jax.numpy as jnp
from jax import lax
from jax.experimental import pallas as pl
from jax.experimental.pallas import tpu as pltpu
from jax.experimental.pallas import tpu_sc as plsc

_N = 10000
_E = 320000
_DIM = 128
_HEADS = 4
_DH = _DIM // _HEADS
_SCALE = _DH ** -0.5

_NC = 2
_NS = 16
_NCHUNK = 1
_EC = _E // _NCHUNK
_EPC = _EC // _NC
_EPT = _EPC // _NS
_C1 = 80
_N1 = _EPT // _C1
_C2 = 80
_N2 = _EPT // _C2
_RPT = 624
_TAIL = _N - _RPT * _NS
_DPT = 2496
_DTAIL = _N * _HEADS - _DPT * _NS

_MESH = plsc.VectorSubcoreMesh(
    core_axis_name="c", subcore_axis_name="s", num_cores=_NC, num_subcores=_NS
)


def _mm_t(x, w):
    return lax.dot_general(x, w, (((1,), (1,)), ((), ())),
                           preferred_element_type=jnp.float32)


def _tc0_body(x_ref, w_ref, b_ref, o_ref):
    o_ref[...] = _mm_t(x_ref[...], w_ref[...]) + b_ref[...]


def _tc0(q_nodes, Wq, bq):
    T = 2000
    return pl.pallas_call(
        _tc0_body,
        grid=(_N // T,),
        in_specs=[
            pl.BlockSpec((T, _DIM), lambda i: (i, 0)),
            pl.BlockSpec((_DIM, _DIM), lambda i: (0, 0)),
            pl.BlockSpec((1, _DIM), lambda i: (0, 0)),
        ],
        out_specs=pl.BlockSpec((T, _DIM), lambda i: (i, 0)),
        out_shape=jax.ShapeDtypeStruct((_N, _DIM), jnp.float32),
    )(q_nodes, Wq, bq.reshape(1, _DIM))


def _make_sc1(e0):
    def body(qp_hbm, dst_hbm, qg_hbm, qp_sh, idx0, idx1, qg0, qg1,
             is0, is1, gs0, gs1, ss0, ss1):
        c = lax.axis_index("c")
        s = lax.axis_index("s")

        @pl.when(s == 0)
        def _():
            pltpu.sync_copy(qp_hbm, qp_sh)

        plsc.subcore_barrier()
        src0 = e0 + c * _EPC + s * _EPT
        out0 = c * _EPC + s * _EPT
        idx = (idx0, idx1)
        qg = (qg0, qg1)
        isem = (is0, is1)
        gsem = (gs0, gs1)
        ssem = (ss0, ss1)

        def step(i, b):
            o = 1 - b
            pltpu.make_async_copy(dst_hbm.at[pl.ds(src0 + i * _C1, _C1)],
                                  idx[b], isem[b]).wait()

            @pl.when(i >= 2)
            def _():
                pltpu.make_async_copy(
                    qg[b], qg_hbm.at[pl.ds(out0 + i * _C1, _C1)],
                    ssem[b]).wait()

            pltpu.async_copy(qp_sh.at[idx[b]], qg[b], gsem[b])

            @pl.when(i >= 1)
            def _():
                pb = out0 + (i - 1) * _C1
                pltpu.make_async_copy(qp_sh.at[idx[o]], qg[o],
                                      gsem[o]).wait()
                pltpu.async_copy(qg[o], qg_hbm.at[pl.ds(pb, _C1)], ssem[o])

                @pl.when(i + 1 < _N1)
                def _():
                    pltpu.async_copy(
                        dst_hbm.at[pl.ds(src0 + (i + 1) * _C1, _C1)],
                        idx[o], isem[o])

        for b in (0, 1):
            pltpu.async_copy(dst_hbm.at[pl.ds(src0 + b * _C1, _C1)], idx[b],
                             isem[b])

        def pair(g, carry):
            for b in (0, 1):
                step(2 * g + b, b)
            return carry

        lax.fori_loop(0, _N1 // 2, pair, 0)
        bl = (_N1 - 1) % 2
        if _N1 % 2:
            step(_N1 - 1, bl)
        lb = out0 + (_N1 - 1) * _C1
        pltpu.make_async_copy(qp_sh.at[idx[bl]], qg[bl], gsem[bl]).wait()
        pltpu.async_copy(qg[bl], qg_hbm.at[pl.ds(lb, _C1)], ssem[bl])
        pltpu.make_async_copy(qg[0], qg_hbm.at[pl.ds(lb, _C1)], ssem[0]).wait()
        pltpu.make_async_copy(qg[1], qg_hbm.at[pl.ds(lb, _C1)], ssem[1]).wait()

    return pl.kernel(
        body,
        out_type=jax.ShapeDtypeStruct((_EC, _DIM), jnp.float32),
        mesh=_MESH,
        scratch_types=[
            pltpu.VMEM_SHARED((_N, _DIM), jnp.float32),
            pltpu.VMEM((_C1,), jnp.int32),
            pltpu.VMEM((_C1,), jnp.int32),
            pltpu.VMEM((_C1, _DIM), jnp.float32),
            pltpu.VMEM((_C1, _DIM), jnp.float32),
        ] + [pltpu.SemaphoreType.DMA] * 6,
    )


def _tc1_body(k_ref, v_ref, qg_ref, wk_ref, bk_ref, wv_ref, bv_ref, m_ref,
              w_ref, u_ref):
    kp = _mm_t(k_ref[...], wk_ref[...]) + bk_ref[...]
    s = jnp.dot(qg_ref[...] * kp, m_ref[...],
                preferred_element_type=jnp.float32) * _SCALE
    u = jnp.exp(s)
    vp = _mm_t(v_ref[...], wv_ref[...]) + bv_ref[...]
    w_ref[...] = vp * _mm_t(u, m_ref[...])
    u_ref[...] = u


def _tc1(k_edges, v_edges, qg, Wk, bk, Wv, bv, mask, e0):
    T = 4000
    off = e0 // T
    full = lambda i: (0, 0)
    gblk = lambda i: (i + off, 0)
    blk = lambda i: (i, 0)
    return pl.pallas_call(
        _tc1_body,
        grid=(_EC // T,),
        in_specs=[
            pl.BlockSpec((T, _DIM), gblk),
            pl.BlockSpec((T, _DIM), gblk),
            pl.BlockSpec((T, _DIM), blk),
            pl.BlockSpec((_DIM, _DIM), full),
            pl.BlockSpec((1, _DIM), full),
            pl.BlockSpec((_DIM, _DIM), full),
            pl.BlockSpec((1, _DIM), full),
            pl.BlockSpec((_DIM, _HEADS), full),
        ],
        out_specs=[
            pl.BlockSpec((T, _DIM), blk),
            pl.BlockSpec((T, _HEADS), blk),
        ],
        out_shape=[
            jax.ShapeDtypeStruct((_EC, _DIM), jnp.float32),
            jax.ShapeDtypeStruct((_EC, _HEADS), jnp.float32),
        ],
    )(k_edges, v_edges, qg, Wk, bk.reshape(1, _DIM), Wv, bv.reshape(1, _DIM),
      mask)


def _make_sc2(e0):
    def body(w_hbm, uf_hbm, dst_hbm, i4_hbm, zag_hbm, zden_hbm, agg_out,
             den_out, agg_sh, den_sh, idx0, idx1, w0, w1, u0, u1, q0, q1,
             d_v, dt_v, ls0, ls1):
        c = lax.axis_index("c")
        s = lax.axis_index("s")
        idx = (idx0, idx1)
        wv = (w0, w1)
        uv = (u0, u1)
        qv = (q0, q1)
        lsem = (ls0, ls1)

        @pl.when(s == 0)
        def _():
            pltpu.sync_copy(zag_hbm, agg_sh)
            pltpu.sync_copy(zden_hbm, den_sh)

        plsc.subcore_barrier()
        src0 = e0 + c * _EPC + s * _EPT
        loc0 = c * _EPC + s * _EPT

        def issue(i, b):
            sb = src0 + i * _C2
            lb = loc0 + i * _C2
            pltpu.async_copy(dst_hbm.at[pl.ds(sb, _C2)], idx[b], lsem[b])
            pltpu.async_copy(w_hbm.at[pl.ds(lb, _C2)], wv[b], lsem[b])
            pltpu.async_copy(uf_hbm.at[pl.ds(lb * _HEADS, _C2 * _HEADS)],
                             uv[b], lsem[b])
            pltpu.async_copy(i4_hbm.at[pl.ds(sb * _HEADS, _C2 * _HEADS)],
                             qv[b], lsem[b])

        def drain(i, b):
            sb = src0 + i * _C2
            lb = loc0 + i * _C2
            pltpu.make_async_copy(dst_hbm.at[pl.ds(sb, _C2)], idx[b],
                                  lsem[b]).wait()
            pltpu.make_async_copy(w_hbm.at[pl.ds(lb, _C2)], wv[b],
                                  lsem[b]).wait()
            pltpu.make_async_copy(
                uf_hbm.at[pl.ds(lb * _HEADS, _C2 * _HEADS)], uv[b],
                lsem[b]).wait()
            pltpu.make_async_copy(
                i4_hbm.at[pl.ds(sb * _HEADS, _C2 * _HEADS)], qv[b],
                lsem[b]).wait()

        def scat(b):
            pltpu.sync_copy(wv[b], agg_sh.at[idx[b]], add=True)
            pltpu.sync_copy(uv[b], den_sh.at[qv[b]], add=True)

        for b in (0, 1):
            issue(b, b)

        def pair(g, carry):
            for b in (0, 1):
                i = 2 * g + b
                drain(i, b)
                scat(b)

                @pl.when(i + 2 < _N2)
                def _():
                    issue(i + 2, b)
            return carry

        lax.fori_loop(0, _N2 // 2, pair, 0)
        if _N2 % 2:
            drain(_N2 - 1, 0)
            scat(0)

        plsc.subcore_barrier()
        r0 = s * _RPT
        pltpu.sync_copy(agg_sh.at[pl.ds(r0, _RPT)],
                        agg_out.at[c, pl.ds(r0, _RPT)])
        d0 = s * _DPT
        pltpu.sync_copy(den_sh.at[pl.ds(d0, _DPT)], d_v)
        pltpu.sync_copy(d_v, den_out.at[pl.ds(c * _N * _HEADS + d0, _DPT)])

        @pl.when(s == 0)
        def _():
            t0 = _RPT * _NS
            pltpu.sync_copy(agg_sh.at[pl.ds(t0, _TAIL)],
                            agg_out.at[c, pl.ds(t0, _TAIL)])
            dt0 = _DPT * _NS
            pltpu.sync_copy(den_sh.at[pl.ds(dt0, _DTAIL)], dt_v)
            pltpu.sync_copy(dt_v,
                            den_out.at[pl.ds(c * _N * _HEADS + dt0, _DTAIL)])

    return pl.kernel(
        body,
        out_type=(
            jax.ShapeDtypeStruct((_NC, _N, _DIM), jnp.float32),
            jax.ShapeDtypeStruct((_NC * _N * _HEADS,), jnp.float32),
        ),
        mesh=_MESH,
        scratch_types=[
            pltpu.VMEM_SHARED((_N, _DIM), jnp.float32),
            pltpu.VMEM_SHARED((_N * _HEADS,), jnp.float32),
            pltpu.VMEM((_C2,), jnp.int32),
            pltpu.VMEM((_C2,), jnp.int32),
            pltpu.VMEM((_C2, _DIM), jnp.float32),
            pltpu.VMEM((_C2, _DIM), jnp.float32),
            pltpu.VMEM((_C2 * _HEADS,), jnp.float32),
            pltpu.VMEM((_C2 * _HEADS,), jnp.float32),
            pltpu.VMEM((_C2 * _HEADS,), jnp.int32),
            pltpu.VMEM((_C2 * _HEADS,), jnp.int32),
            pltpu.VMEM((_DPT,), jnp.float32),
            pltpu.VMEM((_DTAIL,), jnp.float32),
        ] + [pltpu.SemaphoreType.DMA] * 2,
    )


def _tc2_body(a_ref, d_ref, wo_ref, bo_ref, m_ref, o_ref):
    agg = a_ref[0] + a_ref[1]
    den = d_ref[0] + d_ref[1]
    d128 = _mm_t(den, m_ref[...])
    nrm = agg / (d128 + 1e-16)
    o_ref[...] = _mm_t(nrm, wo_ref[...]) + bo_ref[...]


def _tc2(agg, den, Wo, bo, mask):
    B = 1000
    full = lambda i: (0, 0)
    pblk = lambda i: (0, i, 0)
    return pl.pallas_call(
        _tc2_body,
        grid=(_N // B,),
        in_specs=[
            pl.BlockSpec((_NC, B, _DIM), pblk),
            pl.BlockSpec((_NC, B, _HEADS), pblk),
            pl.BlockSpec((_DIM, _DIM), full),
            pl.BlockSpec((1, _DIM), full),
            pl.BlockSpec((_DIM, _HEADS), full),
        ],
        out_specs=pl.BlockSpec((B, _DIM), lambda i: (i, 0)),
        out_shape=jax.ShapeDtypeStruct((_N, _DIM), jnp.float32),
    )(agg, den.reshape(_NC, _N, _HEADS), Wo, bo.reshape(1, _DIM), mask)


def kernel(q_nodes, k_edges, v_edges, edge_index, Wq, bq, Wk, bk, Wv, bv, Wo,
           bo):
    dst = edge_index[0]
    idx4 = dst[:, None] * _HEADS + jnp.arange(_HEADS, dtype=jnp.int32)[None, :]
    idx4 = idx4.reshape(_E * _HEADS)
    zag = jnp.zeros((_N, _DIM), jnp.float32)
    zden = jnp.zeros((_N * _HEADS,), jnp.float32)
    mask = (jnp.arange(_DIM)[:, None] // _DH
            == jnp.arange(_HEADS)[None, :]).astype(jnp.float32)

    qp = _tc0(q_nodes, Wq, bq)
    qg = _make_sc1(0)(qp, dst)
    w, u = _tc1(k_edges, v_edges, qg, Wk, bk, Wv, bv, mask, 0)
    agg, den = _make_sc2(0)(w, u.reshape(_EC * _HEADS), dst, idx4, zag, zden)
    return _tc2(agg, den, Wo, bo, mask)


# speedup vs baseline: 1.0579x; 1.0047x over previous
import jax
import jax.numpy as jnp
from jax import lax
from jax.experimental import pallas as pl
from jax.experimental.pallas import tpu as pltpu
from jax.experimental.pallas import tpu_sc as plsc

_N = 10000
_E = 320000
_DIM = 128
_HEADS = 4
_DH = _DIM // _HEADS
_SCALE = _DH ** -0.5

_NC = 2
_NS = 16
_NCHUNK = 1
_EC = _E // _NCHUNK
_EPC = _EC // _NC
_EPT = _EPC // _NS
_C1 = 80
_N1 = _EPT // _C1
_C2 = 80
_N2 = _EPT // _C2
_RPT = 624
_TAIL = _N - _RPT * _NS
_DPT = 2496
_DTAIL = _N * _HEADS - _DPT * _NS

_MESH = plsc.VectorSubcoreMesh(
    core_axis_name="c", subcore_axis_name="s", num_cores=_NC, num_subcores=_NS
)


def _mm_t(x, w):
    return lax.dot_general(x, w, (((1,), (1,)), ((), ())),
                           preferred_element_type=jnp.float32)


def _tc0_body(x_ref, w_ref, b_ref, o_ref):
    o_ref[...] = _mm_t(x_ref[...], w_ref[...]) + b_ref[...]


def _tc0(q_nodes, Wq, bq):
    T = 2000
    return pl.pallas_call(
        _tc0_body,
        grid=(_N // T,),
        in_specs=[
            pl.BlockSpec((T, _DIM), lambda i: (i, 0)),
            pl.BlockSpec((_DIM, _DIM), lambda i: (0, 0)),
            pl.BlockSpec((1, _DIM), lambda i: (0, 0)),
        ],
        out_specs=pl.BlockSpec((T, _DIM), lambda i: (i, 0)),
        out_shape=jax.ShapeDtypeStruct((_N, _DIM), jnp.float32),
    )(q_nodes, Wq, bq.reshape(1, _DIM))


def _make_sc1(e0):
    def body(qp_hbm, dst_hbm, qg_hbm, qp_sh, idx0, idx1, qg0, qg1,
             is0, is1, gs0, gs1, ss0, ss1):
        c = lax.axis_index("c")
        s = lax.axis_index("s")

        @pl.when(s == 0)
        def _():
            pltpu.sync_copy(qp_hbm, qp_sh)

        plsc.subcore_barrier()
        src0 = e0 + c * _EPC + s * _EPT
        out0 = c * _EPC + s * _EPT
        idx = (idx0, idx1)
        qg = (qg0, qg1)
        isem = (is0, is1)
        gsem = (gs0, gs1)
        ssem = (ss0, ss1)

        def step(i, b):
            o = 1 - b
            pltpu.make_async_copy(dst_hbm.at[pl.ds(src0 + i * _C1, _C1)],
                                  idx[b], isem[b]).wait()

            @pl.when(i >= 2)
            def _():
                pltpu.make_async_copy(
                    qg[b], qg_hbm.at[pl.ds(out0 + i * _C1, _C1)],
                    ssem[b]).wait()

            pltpu.async_copy(qp_sh.at[idx[b]], qg[b], gsem[b])

            @pl.when(i >= 1)
            def _():
                pb = out0 + (i - 1) * _C1
                pltpu.make_async_copy(qp_sh.at[idx[o]], qg[o],
                                      gsem[o]).wait()
                pltpu.async_copy(qg[o], qg_hbm.at[pl.ds(pb, _C1)], ssem[o])

                @pl.when(i + 1 < _N1)
                def _():
                    pltpu.async_copy(
                        dst_hbm.at[pl.ds(src0 + (i + 1) * _C1, _C1)],
                        idx[o], isem[o])

        for b in (0, 1):
            pltpu.async_copy(dst_hbm.at[pl.ds(src0 + b * _C1, _C1)], idx[b],
                             isem[b])

        def pair(g, carry):
            for b in (0, 1):
                step(2 * g + b, b)
            return carry

        lax.fori_loop(0, _N1 // 2, pair, 0)
        bl = (_N1 - 1) % 2
        if _N1 % 2:
            step(_N1 - 1, bl)
        lb = out0 + (_N1 - 1) * _C1
        pltpu.make_async_copy(qp_sh.at[idx[bl]], qg[bl], gsem[bl]).wait()
        pltpu.async_copy(qg[bl], qg_hbm.at[pl.ds(lb, _C1)], ssem[bl])
        pltpu.make_async_copy(qg[0], qg_hbm.at[pl.ds(lb, _C1)], ssem[0]).wait()
        pltpu.make_async_copy(qg[1], qg_hbm.at[pl.ds(lb, _C1)], ssem[1]).wait()

    return pl.kernel(
        body,
        out_type=jax.ShapeDtypeStruct((_EC, _DIM), jnp.float32),
        mesh=_MESH,
        scratch_types=[
            pltpu.VMEM_SHARED((_N, _DIM), jnp.float32),
            pltpu.VMEM((_C1,), jnp.int32),
            pltpu.VMEM((_C1,), jnp.int32),
            pltpu.VMEM((_C1, _DIM), jnp.float32),
            pltpu.VMEM((_C1, _DIM), jnp.float32),
        ] + [pltpu.SemaphoreType.DMA] * 6,
    )


def _tc1_body(k_ref, v_ref, qg_ref, wk_ref, bk_ref, wv_ref, bv_ref, m_ref,
              w_ref, u_ref):
    kp = _mm_t(k_ref[...], wk_ref[...]) + bk_ref[...]
    s = jnp.dot(qg_ref[...] * kp, m_ref[...],
                preferred_element_type=jnp.float32) * _SCALE
    u = jnp.exp(s)
    vp = _mm_t(v_ref[...], wv_ref[...]) + bv_ref[...]
    w_ref[...] = vp * _mm_t(u, m_ref[...])
    u_ref[...] = u


def _tc1(k_edges, v_edges, qg, Wk, bk, Wv, bv, mask, e0):
    T = 8000
    off = e0 // T
    full = lambda i: (0, 0)
    gblk = lambda i: (i + off, 0)
    blk = lambda i: (i, 0)
    return pl.pallas_call(
        _tc1_body,
        grid=(_EC // T,),
        in_specs=[
            pl.BlockSpec((T, _DIM), gblk),
            pl.BlockSpec((T, _DIM), gblk),
            pl.BlockSpec((T, _DIM), blk),
            pl.BlockSpec((_DIM, _DIM), full),
            pl.BlockSpec((1, _DIM), full),
            pl.BlockSpec((_DIM, _DIM), full),
            pl.BlockSpec((1, _DIM), full),
            pl.BlockSpec((_DIM, _HEADS), full),
        ],
        out_specs=[
            pl.BlockSpec((T, _DIM), blk),
            pl.BlockSpec((T, _HEADS), blk),
        ],
        out_shape=[
            jax.ShapeDtypeStruct((_EC, _DIM), jnp.float32),
            jax.ShapeDtypeStruct((_EC, _HEADS), jnp.float32),
        ],
    )(k_edges, v_edges, qg, Wk, bk.reshape(1, _DIM), Wv, bv.reshape(1, _DIM),
      mask)


def _make_sc2(e0):
    def body(w_hbm, uf_hbm, dst_hbm, i4_hbm, zag_hbm, zden_hbm, agg_out,
             den_out, agg_sh, den_sh, idx0, idx1, w0, w1, u0, u1, q0, q1,
             d_v, dt_v, ls0, ls1):
        c = lax.axis_index("c")
        s = lax.axis_index("s")
        idx = (idx0, idx1)
        wv = (w0, w1)
        uv = (u0, u1)
        qv = (q0, q1)
        lsem = (ls0, ls1)

        @pl.when(s == 0)
        def _():
            pltpu.sync_copy(zag_hbm, agg_sh)
            pltpu.sync_copy(zden_hbm, den_sh)

        plsc.subcore_barrier()
        src0 = e0 + c * _EPC + s * _EPT
        loc0 = c * _EPC + s * _EPT

        def issue(i, b):
            sb = src0 + i * _C2
            lb = loc0 + i * _C2
            pltpu.async_copy(dst_hbm.at[pl.ds(sb, _C2)], idx[b], lsem[b])
            pltpu.async_copy(w_hbm.at[pl.ds(lb, _C2)], wv[b], lsem[b])
            pltpu.async_copy(uf_hbm.at[pl.ds(lb * _HEADS, _C2 * _HEADS)],
                             uv[b], lsem[b])
            pltpu.async_copy(i4_hbm.at[pl.ds(sb * _HEADS, _C2 * _HEADS)],
                             qv[b], lsem[b])

        def drain(i, b):
            sb = src0 + i * _C2
            lb = loc0 + i * _C2
            pltpu.make_async_copy(dst_hbm.at[pl.ds(sb, _C2)], idx[b],
                                  lsem[b]).wait()
            pltpu.make_async_copy(w_hbm.at[pl.ds(lb, _C2)], wv[b],
                                  lsem[b]).wait()
            pltpu.make_async_copy(
                uf_hbm.at[pl.ds(lb * _HEADS, _C2 * _HEADS)], uv[b],
                lsem[b]).wait()
            pltpu.make_async_copy(
                i4_hbm.at[pl.ds(sb * _HEADS, _C2 * _HEADS)], qv[b],
                lsem[b]).wait()

        def scat(b):
            pltpu.sync_copy(wv[b], agg_sh.at[idx[b]], add=True)
            pltpu.sync_copy(uv[b], den_sh.at[qv[b]], add=True)

        for b in (0, 1):
            issue(b, b)

        def pair(g, carry):
            for b in (0, 1):
                i = 2 * g + b
                drain(i, b)
                scat(b)

                @pl.when(i + 2 < _N2)
                def _():
                    issue(i + 2, b)
            return carry

        lax.fori_loop(0, _N2 // 2, pair, 0)
        if _N2 % 2:
            drain(_N2 - 1, 0)
            scat(0)

        plsc.subcore_barrier()
        r0 = s * _RPT
        pltpu.sync_copy(agg_sh.at[pl.ds(r0, _RPT)],
                        agg_out.at[c, pl.ds(r0, _RPT)])
        d0 = s * _DPT
        pltpu.sync_copy(den_sh.at[pl.ds(d0, _DPT)], d_v)
        pltpu.sync_copy(d_v, den_out.at[pl.ds(c * _N * _HEADS + d0, _DPT)])

        @pl.when(s == 0)
        def _():
            t0 = _RPT * _NS
            pltpu.sync_copy(agg_sh.at[pl.ds(t0, _TAIL)],
                            agg_out.at[c, pl.ds(t0, _TAIL)])
            dt0 = _DPT * _NS
            pltpu.sync_copy(den_sh.at[pl.ds(dt0, _DTAIL)], dt_v)
            pltpu.sync_copy(dt_v,
                            den_out.at[pl.ds(c * _N * _HEADS + dt0, _DTAIL)])

    return pl.kernel(
        body,
        out_type=(
            jax.ShapeDtypeStruct((_NC, _N, _DIM), jnp.float32),
            jax.ShapeDtypeStruct((_NC * _N * _HEADS,), jnp.float32),
        ),
        mesh=_MESH,
        scratch_types=[
            pltpu.VMEM_SHARED((_N, _DIM), jnp.float32),
            pltpu.VMEM_SHARED((_N * _HEADS,), jnp.float32),
            pltpu.VMEM((_C2,), jnp.int32),
            pltpu.VMEM((_C2,), jnp.int32),
            pltpu.VMEM((_C2, _DIM), jnp.float32),
            pltpu.VMEM((_C2, _DIM), jnp.float32),
            pltpu.VMEM((_C2 * _HEADS,), jnp.float32),
            pltpu.VMEM((_C2 * _HEADS,), jnp.float32),
            pltpu.VMEM((_C2 * _HEADS,), jnp.int32),
            pltpu.VMEM((_C2 * _HEADS,), jnp.int32),
            pltpu.VMEM((_DPT,), jnp.float32),
            pltpu.VMEM((_DTAIL,), jnp.float32),
        ] + [pltpu.SemaphoreType.DMA] * 2,
    )


def _tc2_body(a_ref, d_ref, wo_ref, bo_ref, m_ref, o_ref):
    agg = a_ref[0] + a_ref[1]
    den = d_ref[0] + d_ref[1]
    d128 = _mm_t(den, m_ref[...])
    nrm = agg / (d128 + 1e-16)
    o_ref[...] = _mm_t(nrm, wo_ref[...]) + bo_ref[...]


def _tc2(agg, den, Wo, bo, mask):
    B = 1000
    full = lambda i: (0, 0)
    pblk = lambda i: (0, i, 0)
    return pl.pallas_call(
        _tc2_body,
        grid=(_N // B,),
        in_specs=[
            pl.BlockSpec((_NC, B, _DIM), pblk),
            pl.BlockSpec((_NC, B, _HEADS), pblk),
            pl.BlockSpec((_DIM, _DIM), full),
            pl.BlockSpec((1, _DIM), full),
            pl.BlockSpec((_DIM, _HEADS), full),
        ],
        out_specs=pl.BlockSpec((B, _DIM), lambda i: (i, 0)),
        out_shape=jax.ShapeDtypeStruct((_N, _DIM), jnp.float32),
    )(agg, den.reshape(_NC, _N, _HEADS), Wo, bo.reshape(1, _DIM), mask)


def kernel(q_nodes, k_edges, v_edges, edge_index, Wq, bq, Wk, bk, Wv, bv, Wo,
           bo):
    dst = edge_index[0]
    idx4 = dst[:, None] * _HEADS + jnp.arange(_HEADS, dtype=jnp.int32)[None, :]
    idx4 = idx4.reshape(_E * _HEADS)
    zag = jnp.zeros((_N, _DIM), jnp.float32)
    zden = jnp.zeros((_N * _HEADS,), jnp.float32)
    mask = (jnp.arange(_DIM)[:, None] // _DH
            == jnp.arange(_HEADS)[None, :]).astype(jnp.float32)

    qp = _tc0(q_nodes, Wq, bq)
    qg = _make_sc1(0)(qp, dst)
    w, u = _tc1(k_edges, v_edges, qg, Wk, bk, Wv, bv, mask, 0)
    agg, den = _make_sc2(0)(w, u.reshape(_EC * _HEADS), dst, idx4, zag, zden)
    return _tc2(agg, den, Wo, bo, mask)
